# baseline (speedup 1.0000x reference)
import functools

import jax
import jax.numpy as jnp
from jax import lax
from jax.experimental import pallas as pl
from jax.experimental.pallas import tpu as pltpu
from jax.experimental.pallas import tpu_sc as plsc

F32 = jnp.float32
NC = 2
NS = 16
NW = NC * NS
CSZ = 128
G = 8
ALPHA = 0.1


def _ceil_div(a, b):
  return (a + b - 1) // b


def _make_deg_kernel(chunks, npad):
  mesh = plsc.VectorSubcoreMesh(core_axis_name="c", subcore_axis_name="s")
  rows_per_tile = npad // NS
  ncopy = rows_per_tile // CSZ

  @functools.partial(
      pl.kernel, mesh=mesh,
      out_type=jax.ShapeDtypeStruct((NC, npad), F32),
      scratch_types=[
          pltpu.VMEM((chunks, CSZ), jnp.int32),
          pltpu.VMEM((CSZ,), F32),
          pltpu.VMEM_SHARED((npad,), F32),
      ],
  )
  def deg_kernel(dst_hbm, out_hbm, dst_v, vec_v, acc_sh):
    c = lax.axis_index("c")
    s = lax.axis_index("s")
    pltpu.sync_copy(dst_hbm.at[c, s], dst_v)
    zeros16 = jnp.zeros((16,), F32)
    for j in range(CSZ // 16):
      vec_v[pl.ds(j * 16, 16)] = zeros16
    for k in range(ncopy):
      pltpu.sync_copy(vec_v, acc_sh.at[pl.ds(s * rows_per_tile + k * CSZ, CSZ)])
    plsc.subcore_barrier()
    ones16 = jnp.ones((16,), F32)
    for j in range(CSZ // 16):
      vec_v[pl.ds(j * 16, 16)] = ones16

    def body(k, carry):
      pltpu.sync_copy(vec_v, acc_sh.at[dst_v.at[k]], add=True)
      return carry

    lax.fori_loop(0, chunks, body, 0)
    plsc.subcore_barrier()
    pltpu.sync_copy(acc_sh.at[pl.ds(s * rows_per_tile, rows_per_tile)],
                    out_hbm.at[c, pl.ds(s * rows_per_tile, rows_per_tile)])

  return deg_kernel


def _make_agg_kernel(chunks, npad, d):
  mesh = plsc.VectorSubcoreMesh(core_axis_name="c", subcore_axis_name="s")
  rows_per_tile = npad // NS
  ncopy = rows_per_tile // CSZ
  ngroups = chunks // G

  @functools.partial(
      pl.kernel, mesh=mesh,
      out_type=jax.ShapeDtypeStruct((NC, npad, d), F32),
      scratch_types=[
          pltpu.VMEM((chunks, CSZ), jnp.int32),
          pltpu.VMEM((chunks, CSZ), jnp.int32),
          pltpu.VMEM((CSZ, d), F32),
          pltpu.SemaphoreType.DMA,
          pltpu.VMEM_SHARED((npad, d), F32),
      ],
  )
  def agg_kernel(hp_hbm, src_hbm, dst_hbm, out_hbm,
                 src_v, dst_v, rows0, gsem, acc_sh):
    c = lax.axis_index("c")
    s = lax.axis_index("s")
    pltpu.sync_copy(src_hbm.at[c, s], src_v)
    pltpu.sync_copy(dst_hbm.at[c, s], dst_v)

    zeros16 = jnp.zeros((16,), F32)

    def zbody(i, carry):
      r = i // (d // 16)
      col = (i % (d // 16)) * 16
      rows0[r, pl.ds(col, 16)] = zeros16
      return carry

    lax.fori_loop(0, CSZ * (d // 16), zbody, 0)
    for k in range(ncopy):
      pltpu.sync_copy(rows0, acc_sh.at[pl.ds(s * rows_per_tile + k * CSZ, CSZ)])
    plsc.subcore_barrier()

    def body(k, carry):
      pltpu.async_copy(hp_hbm.at[src_v.at[k]], rows0, gsem).wait()
      pltpu.sync_copy(rows0, acc_sh.at[dst_v.at[k]], add=True)
      return carry

    lax.fori_loop(0, chunks, body, 0)
    plsc.subcore_barrier()
    for k in range(ncopy):
      r0 = s * rows_per_tile + k * CSZ
      pltpu.sync_copy(acc_sh.at[pl.ds(r0, CSZ)], out_hbm.at[c, pl.ds(r0, CSZ)])

  return agg_kernel


def _prep_body(x_ref, wh_ref, bh_ref, deg_ref, x0_ref, hp_ref, dinv_ref):
  deg = deg_ref[:, 0] + deg_ref[:, 1] + 1.0
  dinv = lax.rsqrt(deg)[:, None]
  x0 = jnp.dot(x_ref[...], wh_ref[...], preferred_element_type=F32) + bh_ref[0, :]
  x0_ref[...] = x0
  hp_ref[...] = x0 * dinv
  dinv_ref[...] = jnp.broadcast_to(dinv, x0.shape)


def _layer_body(raw_ref, h_ref, x0_ref, dinv_ref, w_ref, h1_ref, hp1_ref):
  dinv = dinv_ref[...]
  raw = raw_ref[0] + raw_ref[1]
  agg = dinv * (raw + dinv * h_ref[...])
  xmix = (1.0 - ALPHA) * agg + ALPHA * x0_ref[...]
  out = jnp.dot(xmix, w_ref[...], preferred_element_type=F32)
  h1 = jnp.maximum(out, 0.0)
  h1_ref[...] = h1
  hp1_ref[...] = dinv * h1


def _final_body(raw_ref, h_ref, x0_ref, dinv_ref, w_ref, wo_ref, bo_ref, y_ref):
  dinv = dinv_ref[...]
  raw = raw_ref[0] + raw_ref[1]
  agg = dinv * (raw + dinv * h_ref[...])
  xmix = (1.0 - ALPHA) * agg + ALPHA * x0_ref[...]
  out = jnp.dot(xmix, w_ref[...], preferred_element_type=F32)
  logits = jnp.dot(out, wo_ref[...], preferred_element_type=F32) + bo_ref[0, :]
  m = jnp.max(logits, axis=1, keepdims=True)
  lse = jnp.log(jnp.sum(jnp.exp(logits - m), axis=1, keepdims=True)) + m
  y_ref[...] = logits - lse


def kernel(x, edge_index, Wh, bh, W1_0, W1_1, W1_2, W1_3, Wo, bo):
  n, din = x.shape
  dh = Wh.shape[1]
  dout = Wo.shape[1]
  e = edge_index.shape[1]

  rows_per_tile = _ceil_div(n, NS * CSZ) * CSZ
  npad = rows_per_tile * NS
  chunks = _ceil_div(_ceil_div(e, NW), 2 * G * CSZ) * 2 * G
  epad = NW * chunks * CSZ

  pad = epad - e
  src_r = jnp.concatenate(
      [edge_index[0], jnp.zeros((pad,), jnp.int32)]).reshape(NC, NS, chunks, CSZ)
  dst_r = jnp.concatenate(
      [edge_index[1], jnp.full((pad,), n, jnp.int32)]).reshape(NC, NS, chunks, CSZ)
  xp = jnp.concatenate([x, jnp.zeros((npad - n, din), F32)])
  bh2 = bh.reshape(1, dh)
  bo2 = bo.reshape(1, dout)

  deg = _make_deg_kernel(chunks, npad)(dst_r)
  deg_t = deg.T

  R = 1024
  grid = (npad // R,)
  row_spec = pl.BlockSpec((R, din), lambda r: (r, 0))
  full_spec = pl.BlockSpec((din, dh), lambda r: (0, 0))

  x0, hp, dinv = pl.pallas_call(
      _prep_body,
      grid=grid,
      in_specs=[
          row_spec,
          full_spec,
          pl.BlockSpec((1, dh), lambda r: (0, 0)),
          pl.BlockSpec((R, 2), lambda r: (r, 0)),
      ],
      out_specs=[pl.BlockSpec((R, dh), lambda r: (r, 0))] * 3,
      out_shape=[jax.ShapeDtypeStruct((npad, dh), F32)] * 3,
  )(xp, Wh, bh2, deg_t)

  agg_call = _make_agg_kernel(chunks, npad, dh)
  layer_call = pl.pallas_call(
      _layer_body,
      grid=grid,
      in_specs=[
          pl.BlockSpec((NC, R, dh), lambda r: (0, r, 0)),
          pl.BlockSpec((R, dh), lambda r: (r, 0)),
          pl.BlockSpec((R, dh), lambda r: (r, 0)),
          pl.BlockSpec((R, dh), lambda r: (r, 0)),
          pl.BlockSpec((dh, dh), lambda r: (0, 0)),
      ],
      out_specs=[pl.BlockSpec((R, dh), lambda r: (r, 0))] * 2,
      out_shape=[jax.ShapeDtypeStruct((npad, dh), F32)] * 2,
  )

  h = x0
  for w1 in (W1_0, W1_1, W1_2):
    raw = agg_call(hp, src_r, dst_r)
    h, hp = layer_call(raw, h, x0, dinv, w1)

  raw = agg_call(hp, src_r, dst_r)
  y = pl.pallas_call(
      _final_body,
      grid=grid,
      in_specs=[
          pl.BlockSpec((NC, R, dh), lambda r: (0, r, 0)),
          pl.BlockSpec((R, dh), lambda r: (r, 0)),
          pl.BlockSpec((R, dh), lambda r: (r, 0)),
          pl.BlockSpec((R, dh), lambda r: (r, 0)),
          pl.BlockSpec((dh, dh), lambda r: (0, 0)),
          pl.BlockSpec((dh, dout), lambda r: (0, 0)),
          pl.BlockSpec((1, dout), lambda r: (0, 0)),
      ],
      out_specs=pl.BlockSpec((R, dout), lambda r: (r, 0)),
      out_shape=jax.ShapeDtypeStruct((npad, dout), F32),
  )(raw, h, x0, dinv, W1_3, Wo, bo2)

  return y[:n]


# speedup vs baseline: 2.8749x; 2.8749x over previous
import functools

import jax
import jax.numpy as jnp
from jax import lax
from jax.experimental import pallas as pl
from jax.experimental.pallas import tpu as pltpu
from jax.experimental.pallas import tpu_sc as plsc

F32 = jnp.float32
NC = 2
NS = 16
NW = NC * NS
CSZ = 128
G = 8
ALPHA = 0.1


def _ceil_div(a, b):
  return (a + b - 1) // b


def _make_deg_kernel(chunks, npad):
  mesh = plsc.VectorSubcoreMesh(core_axis_name="c", subcore_axis_name="s")
  rows_per_tile = npad // NS
  ncopy = rows_per_tile // CSZ

  @functools.partial(
      pl.kernel, mesh=mesh,
      out_type=jax.ShapeDtypeStruct((NC, npad), F32),
      scratch_types=[
          pltpu.VMEM((chunks, CSZ), jnp.int32),
          pltpu.VMEM((CSZ,), F32),
          pltpu.VMEM_SHARED((npad,), F32),
      ],
  )
  def deg_kernel(dst_hbm, out_hbm, dst_v, vec_v, acc_sh):
    c = lax.axis_index("c")
    s = lax.axis_index("s")
    pltpu.sync_copy(dst_hbm.at[c, s], dst_v)
    zeros16 = jnp.zeros((16,), F32)
    for j in range(CSZ // 16):
      vec_v[pl.ds(j * 16, 16)] = zeros16
    for k in range(ncopy):
      pltpu.sync_copy(vec_v, acc_sh.at[pl.ds(s * rows_per_tile + k * CSZ, CSZ)])
    plsc.subcore_barrier()
    ones16 = jnp.ones((16,), F32)
    for j in range(CSZ // 16):
      vec_v[pl.ds(j * 16, 16)] = ones16

    def body(k, carry):
      pltpu.sync_copy(vec_v, acc_sh.at[dst_v.at[k]], add=True)
      return carry

    lax.fori_loop(0, chunks, body, 0)
    plsc.subcore_barrier()
    pltpu.sync_copy(acc_sh.at[pl.ds(s * rows_per_tile, rows_per_tile)],
                    out_hbm.at[c, pl.ds(s * rows_per_tile, rows_per_tile)])

  return deg_kernel


def _make_agg_kernel(chunks, npad, d):
  mesh = plsc.VectorSubcoreMesh(core_axis_name="c", subcore_axis_name="s")
  rows_per_tile = npad // NS
  ncopy = rows_per_tile // CSZ
  ngroups = chunks // G

  @functools.partial(
      pl.kernel, mesh=mesh,
      out_type=jax.ShapeDtypeStruct((NC, npad, d), F32),
      scratch_types=[
          pltpu.VMEM((chunks, CSZ), jnp.int32),
          pltpu.VMEM((chunks, CSZ), jnp.int32),
          pltpu.VMEM((CSZ, d), F32),
          pltpu.SemaphoreType.DMA,
          pltpu.VMEM_SHARED((npad, d), F32),
      ],
  )
  def agg_kernel(hp_hbm, src_hbm, dst_hbm, out_hbm,
                 src_v, dst_v, rows0, gsem, acc_sh):
    c = lax.axis_index("c")
    s = lax.axis_index("s")
    pltpu.sync_copy(src_hbm.at[c, s], src_v)
    pltpu.sync_copy(dst_hbm.at[c, s], dst_v)

    zeros16 = jnp.zeros((16,), F32)

    def zbody(i, carry):
      r = i // (d // 16)
      col = (i % (d // 16)) * 16
      rows0[r, pl.ds(col, 16)] = zeros16
      return carry

    lax.fori_loop(0, CSZ * (d // 16), zbody, 0)
    for k in range(ncopy):
      pltpu.sync_copy(rows0, acc_sh.at[pl.ds(s * rows_per_tile + k * CSZ, CSZ)])
    plsc.subcore_barrier()

    def body(k, carry):
      pltpu.async_copy(hp_hbm.at[src_v.at[k]], rows0, gsem).wait()
      pltpu.sync_copy(rows0, acc_sh.at[dst_v.at[k]], add=True)
      return carry

    lax.fori_loop(0, chunks, body, 0)
    plsc.subcore_barrier()
    for k in range(ncopy):
      r0 = s * rows_per_tile + k * CSZ
      pltpu.sync_copy(acc_sh.at[pl.ds(r0, CSZ)], out_hbm.at[c, pl.ds(r0, CSZ)])

  return agg_kernel


def _prep_body(x_ref, wh_ref, bh_ref, deg_ref, x0_ref, hp_ref, dinv_ref):
  deg = deg_ref[:, 0] + deg_ref[:, 1] + 1.0
  dinv = lax.rsqrt(deg)[:, None]
  x0 = jnp.dot(x_ref[...], wh_ref[...], preferred_element_type=F32) + bh_ref[0, :]
  x0_ref[...] = x0
  hp_ref[...] = x0 * dinv
  dinv_ref[...] = jnp.broadcast_to(dinv, x0.shape)


def _layer_body(raw_ref, h_ref, x0_ref, dinv_ref, w_ref, h1_ref, hp1_ref):
  dinv = dinv_ref[...]
  raw = raw_ref[0] + raw_ref[1]
  agg = dinv * (raw + dinv * h_ref[...])
  xmix = (1.0 - ALPHA) * agg + ALPHA * x0_ref[...]
  out = jnp.dot(xmix, w_ref[...], preferred_element_type=F32)
  h1 = jnp.maximum(out, 0.0)
  h1_ref[...] = h1
  hp1_ref[...] = dinv * h1


def _final_body(raw_ref, h_ref, x0_ref, dinv_ref, w_ref, wo_ref, bo_ref, y_ref):
  dinv = dinv_ref[...]
  raw = raw_ref[0] + raw_ref[1]
  agg = dinv * (raw + dinv * h_ref[...])
  xmix = (1.0 - ALPHA) * agg + ALPHA * x0_ref[...]
  out = jnp.dot(xmix, w_ref[...], preferred_element_type=F32)
  logits = jnp.dot(out, wo_ref[...], preferred_element_type=F32) + bo_ref[0, :]
  m = jnp.max(logits, axis=1, keepdims=True)
  lse = jnp.log(jnp.sum(jnp.exp(logits - m), axis=1, keepdims=True)) + m
  y_ref[...] = logits - lse


def kernel(x, edge_index, Wh, bh, W1_0, W1_1, W1_2, W1_3, Wo, bo):
  n, din = x.shape
  dh = Wh.shape[1]
  dout = Wo.shape[1]
  e = edge_index.shape[1]

  rows_per_tile = _ceil_div(n, NS * CSZ) * CSZ
  npad = rows_per_tile * NS
  chunks = _ceil_div(_ceil_div(e, NW), 2 * G * CSZ) * 2 * G
  epad = NW * chunks * CSZ

  pad = epad - e
  pad_iota = jnp.arange(pad, dtype=jnp.int32)
  src_r = jnp.concatenate(
      [edge_index[0], pad_iota % n]).reshape(NC, NS, chunks, CSZ)
  dst_r = jnp.concatenate(
      [edge_index[1], n + pad_iota % (npad - n)]).reshape(NC, NS, chunks, CSZ)
  xp = jnp.concatenate([x, jnp.zeros((npad - n, din), F32)])
  bh2 = bh.reshape(1, dh)
  bo2 = bo.reshape(1, dout)

  deg = _make_deg_kernel(chunks, npad)(dst_r)
  deg_t = deg.T

  R = 1024
  grid = (npad // R,)
  row_spec = pl.BlockSpec((R, din), lambda r: (r, 0))
  full_spec = pl.BlockSpec((din, dh), lambda r: (0, 0))

  x0, hp, dinv = pl.pallas_call(
      _prep_body,
      grid=grid,
      in_specs=[
          row_spec,
          full_spec,
          pl.BlockSpec((1, dh), lambda r: (0, 0)),
          pl.BlockSpec((R, 2), lambda r: (r, 0)),
      ],
      out_specs=[pl.BlockSpec((R, dh), lambda r: (r, 0))] * 3,
      out_shape=[jax.ShapeDtypeStruct((npad, dh), F32)] * 3,
  )(xp, Wh, bh2, deg_t)

  agg_call = _make_agg_kernel(chunks, npad, dh)
  layer_call = pl.pallas_call(
      _layer_body,
      grid=grid,
      in_specs=[
          pl.BlockSpec((NC, R, dh), lambda r: (0, r, 0)),
          pl.BlockSpec((R, dh), lambda r: (r, 0)),
          pl.BlockSpec((R, dh), lambda r: (r, 0)),
          pl.BlockSpec((R, dh), lambda r: (r, 0)),
          pl.BlockSpec((dh, dh), lambda r: (0, 0)),
      ],
      out_specs=[pl.BlockSpec((R, dh), lambda r: (r, 0))] * 2,
      out_shape=[jax.ShapeDtypeStruct((npad, dh), F32)] * 2,
  )

  h = x0
  for w1 in (W1_0, W1_1, W1_2):
    raw = agg_call(hp, src_r, dst_r)
    h, hp = layer_call(raw, h, x0, dinv, w1)

  raw = agg_call(hp, src_r, dst_r)
  y = pl.pallas_call(
      _final_body,
      grid=grid,
      in_specs=[
          pl.BlockSpec((NC, R, dh), lambda r: (0, r, 0)),
          pl.BlockSpec((R, dh), lambda r: (r, 0)),
          pl.BlockSpec((R, dh), lambda r: (r, 0)),
          pl.BlockSpec((R, dh), lambda r: (r, 0)),
          pl.BlockSpec((dh, dh), lambda r: (0, 0)),
          pl.BlockSpec((dh, dout), lambda r: (0, 0)),
          pl.BlockSpec((1, dout), lambda r: (0, 0)),
      ],
      out_specs=pl.BlockSpec((R, dout), lambda r: (r, 0)),
      out_shape=jax.ShapeDtypeStruct((npad, dout), F32),
  )(raw, h, x0, dinv, W1_3, Wo, bo2)

  return y[:n]


# speedup vs baseline: 3.2003x; 1.1132x over previous
import functools

import jax
import jax.numpy as jnp
from jax import lax
from jax.experimental import pallas as pl
from jax.experimental.pallas import tpu as pltpu
from jax.experimental.pallas import tpu_sc as plsc

F32 = jnp.float32
NC = 2
NS = 16
NW = NC * NS
CSZ = 128
G = 8
ALPHA = 0.1


def _ceil_div(a, b):
  return (a + b - 1) // b


def _make_deg_kernel(chunks, npad):
  mesh = plsc.VectorSubcoreMesh(core_axis_name="c", subcore_axis_name="s")
  rows_per_tile = npad // NS
  ncopy = rows_per_tile // CSZ

  @functools.partial(
      pl.kernel, mesh=mesh,
      out_type=jax.ShapeDtypeStruct((NC, npad), F32),
      scratch_types=[
          pltpu.VMEM((chunks, CSZ), jnp.int32),
          pltpu.VMEM((CSZ,), F32),
          pltpu.VMEM_SHARED((npad,), F32),
      ],
  )
  def deg_kernel(dst_hbm, out_hbm, dst_v, vec_v, acc_sh):
    c = lax.axis_index("c")
    s = lax.axis_index("s")
    pltpu.sync_copy(dst_hbm.at[c, s], dst_v)
    zeros16 = jnp.zeros((16,), F32)
    for j in range(CSZ // 16):
      vec_v[pl.ds(j * 16, 16)] = zeros16
    for k in range(ncopy):
      pltpu.sync_copy(vec_v, acc_sh.at[pl.ds(s * rows_per_tile + k * CSZ, CSZ)])
    plsc.subcore_barrier()
    ones16 = jnp.ones((16,), F32)
    for j in range(CSZ // 16):
      vec_v[pl.ds(j * 16, 16)] = ones16

    def body(k, carry):
      pltpu.sync_copy(vec_v, acc_sh.at[dst_v.at[k]], add=True)
      return carry

    lax.fori_loop(0, chunks, body, 0)
    plsc.subcore_barrier()
    pltpu.sync_copy(acc_sh.at[pl.ds(s * rows_per_tile, rows_per_tile)],
                    out_hbm.at[c, pl.ds(s * rows_per_tile, rows_per_tile)])

  return deg_kernel


def _make_agg_kernel(chunks, npad, d):
  mesh = plsc.VectorSubcoreMesh(core_axis_name="c", subcore_axis_name="s")
  rows_per_tile = npad // NS
  ncopy = rows_per_tile // CSZ
  ngroups = chunks // G

  @functools.partial(
      pl.kernel, mesh=mesh,
      out_type=jax.ShapeDtypeStruct((NC, npad, d), F32),
      scratch_types=[
          pltpu.VMEM((chunks, CSZ), jnp.int32),
          pltpu.VMEM((G, CSZ), jnp.int32),
          pltpu.VMEM((CSZ, d), F32),
          pltpu.VMEM((CSZ, d), F32),
          pltpu.SemaphoreType.DMA,
          pltpu.SemaphoreType.DMA,
          pltpu.VMEM_SHARED((npad, d), F32),
      ],
  )
  def agg_kernel(hp_hbm, src_hbm, dst_hbm, out_hbm,
                 src_v, dib, rows0, rows1, gsem0, gsem1, acc_sh):
    c = lax.axis_index("c")
    s = lax.axis_index("s")
    pltpu.sync_copy(src_hbm.at[c, s], src_v)

    zeros16 = jnp.zeros((16,), F32)

    def zbody(i, carry):
      r = i // (d // 16)
      col = (i % (d // 16)) * 16
      rows0[r, pl.ds(col, 16)] = zeros16
      return carry

    lax.fori_loop(0, CSZ * (d // 16), zbody, 0)
    for k in range(ncopy):
      pltpu.sync_copy(rows0, acc_sh.at[pl.ds(s * rows_per_tile + k * CSZ, CSZ)])
    plsc.subcore_barrier()

    def body(g, carry):
      pltpu.sync_copy(dst_hbm.at[c, s, pl.ds(g * G, G)], dib)
      for jp in range(G // 2):
        j0 = 2 * jp
        k0 = g * G + j0
        da = pltpu.async_copy(hp_hbm.at[src_v.at[k0]], rows0, gsem0)
        db = pltpu.async_copy(hp_hbm.at[src_v.at[k0 + 1]], rows1, gsem1)
        da.wait()
        pltpu.sync_copy(rows0, acc_sh.at[dib.at[j0]], add=True)
        db.wait()
        pltpu.sync_copy(rows1, acc_sh.at[dib.at[j0 + 1]], add=True)
      return carry

    lax.fori_loop(0, ngroups, body, 0)
    plsc.subcore_barrier()
    for k in range(ncopy):
      r0 = s * rows_per_tile + k * CSZ
      pltpu.sync_copy(acc_sh.at[pl.ds(r0, CSZ)], out_hbm.at[c, pl.ds(r0, CSZ)])

  return agg_kernel


def _prep_body(x_ref, wh_ref, bh_ref, deg_ref, x0_ref, hp_ref, dinv_ref):
  deg = deg_ref[:, 0] + deg_ref[:, 1] + 1.0
  dinv = lax.rsqrt(deg)[:, None]
  x0 = jnp.dot(x_ref[...], wh_ref[...], preferred_element_type=F32) + bh_ref[0, :]
  x0_ref[...] = x0
  hp_ref[...] = x0 * dinv
  dinv_ref[...] = jnp.broadcast_to(dinv, x0.shape)


def _layer_body(raw_ref, h_ref, x0_ref, dinv_ref, w_ref, h1_ref, hp1_ref):
  dinv = dinv_ref[...]
  raw = raw_ref[0] + raw_ref[1]
  agg = dinv * (raw + dinv * h_ref[...])
  xmix = (1.0 - ALPHA) * agg + ALPHA * x0_ref[...]
  out = jnp.dot(xmix, w_ref[...], preferred_element_type=F32)
  h1 = jnp.maximum(out, 0.0)
  h1_ref[...] = h1
  hp1_ref[...] = dinv * h1


def _final_body(raw_ref, h_ref, x0_ref, dinv_ref, w_ref, wo_ref, bo_ref, y_ref):
  dinv = dinv_ref[...]
  raw = raw_ref[0] + raw_ref[1]
  agg = dinv * (raw + dinv * h_ref[...])
  xmix = (1.0 - ALPHA) * agg + ALPHA * x0_ref[...]
  out = jnp.dot(xmix, w_ref[...], preferred_element_type=F32)
  logits = jnp.dot(out, wo_ref[...], preferred_element_type=F32) + bo_ref[0, :]
  m = jnp.max(logits, axis=1, keepdims=True)
  lse = jnp.log(jnp.sum(jnp.exp(logits - m), axis=1, keepdims=True)) + m
  y_ref[...] = logits - lse


def kernel(x, edge_index, Wh, bh, W1_0, W1_1, W1_2, W1_3, Wo, bo):
  n, din = x.shape
  dh = Wh.shape[1]
  dout = Wo.shape[1]
  e = edge_index.shape[1]

  rows_per_tile = _ceil_div(n, NS * CSZ) * CSZ
  npad = rows_per_tile * NS
  chunks = _ceil_div(_ceil_div(e, NW), 2 * G * CSZ) * 2 * G
  epad = NW * chunks * CSZ

  pad = epad - e
  pad_iota = jnp.arange(pad, dtype=jnp.int32)
  src_r = jnp.concatenate(
      [edge_index[0], pad_iota % n]).reshape(NC, NS, chunks, CSZ)
  dst_r = jnp.concatenate(
      [edge_index[1], n + pad_iota % (npad - n)]).reshape(NC, NS, chunks, CSZ)
  xp = jnp.concatenate([x, jnp.zeros((npad - n, din), F32)])
  bh2 = bh.reshape(1, dh)
  bo2 = bo.reshape(1, dout)

  deg = _make_deg_kernel(chunks, npad)(dst_r)
  deg_t = deg.T

  R = 1024
  grid = (npad // R,)
  row_spec = pl.BlockSpec((R, din), lambda r: (r, 0))
  full_spec = pl.BlockSpec((din, dh), lambda r: (0, 0))

  x0, hp, dinv = pl.pallas_call(
      _prep_body,
      grid=grid,
      in_specs=[
          row_spec,
          full_spec,
          pl.BlockSpec((1, dh), lambda r: (0, 0)),
          pl.BlockSpec((R, 2), lambda r: (r, 0)),
      ],
      out_specs=[pl.BlockSpec((R, dh), lambda r: (r, 0))] * 3,
      out_shape=[jax.ShapeDtypeStruct((npad, dh), F32)] * 3,
  )(xp, Wh, bh2, deg_t)

  agg_call = _make_agg_kernel(chunks, npad, dh)
  layer_call = pl.pallas_call(
      _layer_body,
      grid=grid,
      in_specs=[
          pl.BlockSpec((NC, R, dh), lambda r: (0, r, 0)),
          pl.BlockSpec((R, dh), lambda r: (r, 0)),
          pl.BlockSpec((R, dh), lambda r: (r, 0)),
          pl.BlockSpec((R, dh), lambda r: (r, 0)),
          pl.BlockSpec((dh, dh), lambda r: (0, 0)),
      ],
      out_specs=[pl.BlockSpec((R, dh), lambda r: (r, 0))] * 2,
      out_shape=[jax.ShapeDtypeStruct((npad, dh), F32)] * 2,
  )

  h = x0
  for w1 in (W1_0, W1_1, W1_2):
    raw = agg_call(hp, src_r, dst_r)
    h, hp = layer_call(raw, h, x0, dinv, w1)

  raw = agg_call(hp, src_r, dst_r)
  y = pl.pallas_call(
      _final_body,
      grid=grid,
      in_specs=[
          pl.BlockSpec((NC, R, dh), lambda r: (0, r, 0)),
          pl.BlockSpec((R, dh), lambda r: (r, 0)),
          pl.BlockSpec((R, dh), lambda r: (r, 0)),
          pl.BlockSpec((R, dh), lambda r: (r, 0)),
          pl.BlockSpec((dh, dh), lambda r: (0, 0)),
          pl.BlockSpec((dh, dout), lambda r: (0, 0)),
          pl.BlockSpec((1, dout), lambda r: (0, 0)),
      ],
      out_specs=pl.BlockSpec((R, dout), lambda r: (r, 0)),
      out_shape=jax.ShapeDtypeStruct((npad, dout), F32),
  )(raw, h, x0, dinv, W1_3, Wo, bo2)

  return y[:n]


# speedup vs baseline: 3.2375x; 1.0116x over previous
import functools

import jax
import jax.numpy as jnp
from jax import lax
from jax.experimental import pallas as pl
from jax.experimental.pallas import tpu as pltpu
from jax.experimental.pallas import tpu_sc as plsc

F32 = jnp.float32
NC = 2
NS = 16
NW = NC * NS
CSZ = 128
G = 8
ALPHA = 0.1


def _ceil_div(a, b):
  return (a + b - 1) // b


def _make_deg_kernel(chunks, npad):
  mesh = plsc.VectorSubcoreMesh(core_axis_name="c", subcore_axis_name="s")
  rows_per_tile = npad // NS
  ncopy = rows_per_tile // CSZ

  @functools.partial(
      pl.kernel, mesh=mesh,
      out_type=jax.ShapeDtypeStruct((NC, npad), F32),
      scratch_types=[
          pltpu.VMEM((chunks, CSZ), jnp.int32),
          pltpu.VMEM((CSZ,), F32),
          pltpu.VMEM_SHARED((npad,), F32),
      ],
  )
  def deg_kernel(dst_hbm, out_hbm, dst_v, vec_v, acc_sh):
    c = lax.axis_index("c")
    s = lax.axis_index("s")
    pltpu.sync_copy(dst_hbm.at[c, s], dst_v)
    zeros16 = jnp.zeros((16,), F32)
    for j in range(CSZ // 16):
      vec_v[pl.ds(j * 16, 16)] = zeros16
    for k in range(ncopy):
      pltpu.sync_copy(vec_v, acc_sh.at[pl.ds(s * rows_per_tile + k * CSZ, CSZ)])
    plsc.subcore_barrier()
    ones16 = jnp.ones((16,), F32)
    for j in range(CSZ // 16):
      vec_v[pl.ds(j * 16, 16)] = ones16

    def body(k, carry):
      pltpu.sync_copy(vec_v, acc_sh.at[dst_v.at[k]], add=True)
      return carry

    lax.fori_loop(0, chunks, body, 0)
    plsc.subcore_barrier()
    pltpu.sync_copy(acc_sh.at[pl.ds(s * rows_per_tile, rows_per_tile)],
                    out_hbm.at[c, pl.ds(s * rows_per_tile, rows_per_tile)])

  return deg_kernel


def _make_agg_kernel(chunks, npad, d):
  mesh = plsc.VectorSubcoreMesh(core_axis_name="c", subcore_axis_name="s")
  rows_per_tile = npad // NS
  ncopy = rows_per_tile // CSZ
  ngroups = chunks // G

  @functools.partial(
      pl.kernel, mesh=mesh,
      out_type=jax.ShapeDtypeStruct((NC, npad, d), F32),
      scratch_types=[
          pltpu.VMEM((chunks, CSZ), jnp.int32),
          pltpu.VMEM((G, CSZ), jnp.int32),
          pltpu.VMEM((CSZ, d), F32),
          pltpu.VMEM((CSZ, d), F32),
          pltpu.SemaphoreType.DMA,
          pltpu.SemaphoreType.DMA,
          pltpu.SemaphoreType.DMA,
          pltpu.SemaphoreType.DMA,
          pltpu.VMEM_SHARED((npad, d), F32),
      ],
  )
  def agg_kernel(hp_hbm, src_hbm, dst_hbm, out_hbm,
                 src_v, dib, rows0, rows1, gsem0, gsem1, ssem0, ssem1, acc_sh):
    c = lax.axis_index("c")
    s = lax.axis_index("s")
    pltpu.sync_copy(src_hbm.at[c, s], src_v)

    zeros16 = jnp.zeros((16,), F32)

    def zbody(i, carry):
      r = i // (d // 16)
      col = (i % (d // 16)) * 16
      rows0[r, pl.ds(col, 16)] = zeros16
      return carry

    lax.fori_loop(0, CSZ * (d // 16), zbody, 0)
    for k in range(ncopy):
      pltpu.sync_copy(rows0, acc_sh.at[pl.ds(s * rows_per_tile + k * CSZ, CSZ)])
    plsc.subcore_barrier()

    def body(g, carry):
      pltpu.sync_copy(dst_hbm.at[c, s, pl.ds(g * G, G)], dib)
      for jp in range(G // 2):
        j0 = 2 * jp
        k0 = g * G + j0
        da = pltpu.async_copy(hp_hbm.at[src_v.at[k0]], rows0, gsem0)
        db = pltpu.async_copy(hp_hbm.at[src_v.at[k0 + 1]], rows1, gsem1)
        da.wait()
        sa = pltpu.async_copy(rows0, acc_sh.at[dib.at[j0]], ssem0, add=True)
        db.wait()
        sb = pltpu.async_copy(rows1, acc_sh.at[dib.at[j0 + 1]], ssem1, add=True)
        sa.wait()
        sb.wait()
      return carry

    lax.fori_loop(0, ngroups, body, 0)
    plsc.subcore_barrier()
    for k in range(ncopy):
      r0 = s * rows_per_tile + k * CSZ
      pltpu.sync_copy(acc_sh.at[pl.ds(r0, CSZ)], out_hbm.at[c, pl.ds(r0, CSZ)])

  return agg_kernel


def _prep_body(x_ref, wh_ref, bh_ref, deg_ref, x0_ref, hp_ref, dinv_ref):
  deg = deg_ref[:, 0] + deg_ref[:, 1] + 1.0
  dinv = lax.rsqrt(deg)[:, None]
  x0 = jnp.dot(x_ref[...], wh_ref[...], preferred_element_type=F32) + bh_ref[0, :]
  x0_ref[...] = x0
  hp_ref[...] = x0 * dinv
  dinv_ref[...] = jnp.broadcast_to(dinv, x0.shape)


def _layer_body(raw_ref, h_ref, x0_ref, dinv_ref, w_ref, h1_ref, hp1_ref):
  dinv = dinv_ref[...]
  raw = raw_ref[0] + raw_ref[1]
  agg = dinv * (raw + dinv * h_ref[...])
  xmix = (1.0 - ALPHA) * agg + ALPHA * x0_ref[...]
  out = jnp.dot(xmix, w_ref[...], preferred_element_type=F32)
  h1 = jnp.maximum(out, 0.0)
  h1_ref[...] = h1
  hp1_ref[...] = dinv * h1


def _final_body(raw_ref, h_ref, x0_ref, dinv_ref, w_ref, wo_ref, bo_ref, y_ref):
  dinv = dinv_ref[...]
  raw = raw_ref[0] + raw_ref[1]
  agg = dinv * (raw + dinv * h_ref[...])
  xmix = (1.0 - ALPHA) * agg + ALPHA * x0_ref[...]
  out = jnp.dot(xmix, w_ref[...], preferred_element_type=F32)
  logits = jnp.dot(out, wo_ref[...], preferred_element_type=F32) + bo_ref[0, :]
  m = jnp.max(logits, axis=1, keepdims=True)
  lse = jnp.log(jnp.sum(jnp.exp(logits - m), axis=1, keepdims=True)) + m
  y_ref[...] = logits - lse


def kernel(x, edge_index, Wh, bh, W1_0, W1_1, W1_2, W1_3, Wo, bo):
  n, din = x.shape
  dh = Wh.shape[1]
  dout = Wo.shape[1]
  e = edge_index.shape[1]

  rows_per_tile = _ceil_div(n, NS * CSZ) * CSZ
  npad = rows_per_tile * NS
  chunks = _ceil_div(_ceil_div(e, NW), 2 * G * CSZ) * 2 * G
  epad = NW * chunks * CSZ

  pad = epad - e
  pad_iota = jnp.arange(pad, dtype=jnp.int32)
  src_r = jnp.concatenate(
      [edge_index[0], pad_iota % n]).reshape(NC, NS, chunks, CSZ)
  dst_r = jnp.concatenate(
      [edge_index[1], n + pad_iota % (npad - n)]).reshape(NC, NS, chunks, CSZ)
  xp = jnp.concatenate([x, jnp.zeros((npad - n, din), F32)])
  bh2 = bh.reshape(1, dh)
  bo2 = bo.reshape(1, dout)

  deg = _make_deg_kernel(chunks, npad)(dst_r)
  deg_t = deg.T

  R = 1024
  grid = (npad // R,)
  row_spec = pl.BlockSpec((R, din), lambda r: (r, 0))
  full_spec = pl.BlockSpec((din, dh), lambda r: (0, 0))

  x0, hp, dinv = pl.pallas_call(
      _prep_body,
      grid=grid,
      in_specs=[
          row_spec,
          full_spec,
          pl.BlockSpec((1, dh), lambda r: (0, 0)),
          pl.BlockSpec((R, 2), lambda r: (r, 0)),
      ],
      out_specs=[pl.BlockSpec((R, dh), lambda r: (r, 0))] * 3,
      out_shape=[jax.ShapeDtypeStruct((npad, dh), F32)] * 3,
  )(xp, Wh, bh2, deg_t)

  agg_call = _make_agg_kernel(chunks, npad, dh)
  layer_call = pl.pallas_call(
      _layer_body,
      grid=grid,
      in_specs=[
          pl.BlockSpec((NC, R, dh), lambda r: (0, r, 0)),
          pl.BlockSpec((R, dh), lambda r: (r, 0)),
          pl.BlockSpec((R, dh), lambda r: (r, 0)),
          pl.BlockSpec((R, dh), lambda r: (r, 0)),
          pl.BlockSpec((dh, dh), lambda r: (0, 0)),
      ],
      out_specs=[pl.BlockSpec((R, dh), lambda r: (r, 0))] * 2,
      out_shape=[jax.ShapeDtypeStruct((npad, dh), F32)] * 2,
  )

  h = x0
  for w1 in (W1_0, W1_1, W1_2):
    raw = agg_call(hp, src_r, dst_r)
    h, hp = layer_call(raw, h, x0, dinv, w1)

  raw = agg_call(hp, src_r, dst_r)
  y = pl.pallas_call(
      _final_body,
      grid=grid,
      in_specs=[
          pl.BlockSpec((NC, R, dh), lambda r: (0, r, 0)),
          pl.BlockSpec((R, dh), lambda r: (r, 0)),
          pl.BlockSpec((R, dh), lambda r: (r, 0)),
          pl.BlockSpec((R, dh), lambda r: (r, 0)),
          pl.BlockSpec((dh, dh), lambda r: (0, 0)),
          pl.BlockSpec((dh, dout), lambda r: (0, 0)),
          pl.BlockSpec((1, dout), lambda r: (0, 0)),
      ],
      out_specs=pl.BlockSpec((R, dout), lambda r: (r, 0)),
      out_shape=jax.ShapeDtypeStruct((npad, dout), F32),
  )(raw, h, x0, dinv, W1_3, Wo, bo2)

  return y[:n]


# speedup vs baseline: 4.1402x; 1.2788x over previous
import functools

import jax
import jax.numpy as jnp
from jax import lax
from jax.experimental import pallas as pl
from jax.experimental.pallas import tpu as pltpu
from jax.experimental.pallas import tpu_sc as plsc

F32 = jnp.float32
NC = 2
NS = 16
NW = NC * NS
CSZ = 128
G = 8
ALPHA = 0.1


def _ceil_div(a, b):
  return (a + b - 1) // b


def _make_deg_kernel(chunks, npad):
  mesh = plsc.VectorSubcoreMesh(core_axis_name="c", subcore_axis_name="s")
  rows_per_tile = npad // NS
  ncopy = rows_per_tile // CSZ

  @functools.partial(
      pl.kernel, mesh=mesh,
      out_type=jax.ShapeDtypeStruct((NC, npad), F32),
      scratch_types=[
          pltpu.VMEM((chunks, CSZ), jnp.int32),
          pltpu.VMEM((CSZ,), F32),
          pltpu.VMEM_SHARED((npad,), F32),
      ],
  )
  def deg_kernel(dst_hbm, out_hbm, dst_v, vec_v, acc_sh):
    c = lax.axis_index("c")
    s = lax.axis_index("s")
    pltpu.sync_copy(dst_hbm.at[c, s], dst_v)
    zeros16 = jnp.zeros((16,), F32)
    for j in range(CSZ // 16):
      vec_v[pl.ds(j * 16, 16)] = zeros16
    for k in range(ncopy):
      pltpu.sync_copy(vec_v, acc_sh.at[pl.ds(s * rows_per_tile + k * CSZ, CSZ)])
    plsc.subcore_barrier()
    ones16 = jnp.ones((16,), F32)
    for j in range(CSZ // 16):
      vec_v[pl.ds(j * 16, 16)] = ones16

    def body(k, carry):
      pltpu.sync_copy(vec_v, acc_sh.at[dst_v.at[k]], add=True)
      return carry

    lax.fori_loop(0, chunks, body, 0)
    plsc.subcore_barrier()
    pltpu.sync_copy(acc_sh.at[pl.ds(s * rows_per_tile, rows_per_tile)],
                    out_hbm.at[c, pl.ds(s * rows_per_tile, rows_per_tile)])

  return deg_kernel


def _make_agg_kernel(chunks, npad, d):
  mesh = plsc.VectorSubcoreMesh(core_axis_name="c", subcore_axis_name="s")
  rows_per_tile = npad // NS
  ncopy = rows_per_tile // CSZ
  ngroups = chunks // G

  @functools.partial(
      pl.kernel, mesh=mesh,
      out_type=jax.ShapeDtypeStruct((NC, npad, d), F32),
      scratch_types=[
          pltpu.VMEM((chunks, CSZ), jnp.int32),
          pltpu.VMEM((G, CSZ), jnp.int32),
          pltpu.VMEM((CSZ, d), F32),
          pltpu.VMEM((CSZ, d), F32),
          pltpu.SemaphoreType.DMA,
          pltpu.SemaphoreType.DMA,
          pltpu.SemaphoreType.DMA,
          pltpu.SemaphoreType.DMA,
          pltpu.VMEM_SHARED((npad, d), F32),
      ],
  )
  def agg_kernel(hp_hbm, src_hbm, dst_hbm, out_hbm,
                 src_v, dib, rows0, rows1, gsem0, gsem1, ssem0, ssem1, acc_sh):
    c = lax.axis_index("c")
    s = lax.axis_index("s")
    pltpu.sync_copy(src_hbm.at[c, s], src_v)

    zeros16 = jnp.zeros((16,), F32)

    def zbody(i, carry):
      r = i // (d // 16)
      col = (i % (d // 16)) * 16
      rows0[r, pl.ds(col, 16)] = zeros16
      return carry

    lax.fori_loop(0, CSZ * (d // 16), zbody, 0)
    for k in range(ncopy):
      pltpu.sync_copy(rows0, acc_sh.at[pl.ds(s * rows_per_tile + k * CSZ, CSZ)])
    plsc.subcore_barrier()

    rows = (rows0, rows1)
    gsem = (gsem0, gsem1)
    ssem = (ssem0, ssem1)
    dg = [None, None]
    ds = [None, None]
    dg[0] = pltpu.async_copy(hp_hbm.at[src_v.at[0]], rows0, gsem0)
    for k in range(chunks):
      p = k % 2
      if k >= 1:
        ds[1 - p].wait()
      if k % G == 0:
        pltpu.sync_copy(dst_hbm.at[c, s, pl.ds(k, G)], dib)
      if k + 1 < chunks:
        dg[1 - p] = pltpu.async_copy(
            hp_hbm.at[src_v.at[k + 1]], rows[1 - p], gsem[1 - p])
      dg[p].wait()
      ds[p] = pltpu.async_copy(
          rows[p], acc_sh.at[dib.at[k % G]], ssem[p], add=True)
    ds[(chunks - 1) % 2].wait()
    plsc.subcore_barrier()
    for k in range(ncopy):
      r0 = s * rows_per_tile + k * CSZ
      pltpu.sync_copy(acc_sh.at[pl.ds(r0, CSZ)], out_hbm.at[c, pl.ds(r0, CSZ)])

  return agg_kernel


def _prep_body(x_ref, wh_ref, bh_ref, deg_ref, x0_ref, hp_ref, dinv_ref):
  deg = deg_ref[:, 0] + deg_ref[:, 1] + 1.0
  dinv = lax.rsqrt(deg)[:, None]
  x0 = jnp.dot(x_ref[...], wh_ref[...], preferred_element_type=F32) + bh_ref[0, :]
  x0_ref[...] = x0
  hp_ref[...] = x0 * dinv
  dinv_ref[...] = jnp.broadcast_to(dinv, x0.shape)


def _layer_body(raw_ref, h_ref, x0_ref, dinv_ref, w_ref, h1_ref, hp1_ref):
  dinv = dinv_ref[...]
  raw = raw_ref[0] + raw_ref[1]
  agg = dinv * (raw + dinv * h_ref[...])
  xmix = (1.0 - ALPHA) * agg + ALPHA * x0_ref[...]
  out = jnp.dot(xmix, w_ref[...], preferred_element_type=F32)
  h1 = jnp.maximum(out, 0.0)
  h1_ref[...] = h1
  hp1_ref[...] = dinv * h1


def _final_body(raw_ref, h_ref, x0_ref, dinv_ref, w_ref, wo_ref, bo_ref, y_ref):
  dinv = dinv_ref[...]
  raw = raw_ref[0] + raw_ref[1]
  agg = dinv * (raw + dinv * h_ref[...])
  xmix = (1.0 - ALPHA) * agg + ALPHA * x0_ref[...]
  out = jnp.dot(xmix, w_ref[...], preferred_element_type=F32)
  logits = jnp.dot(out, wo_ref[...], preferred_element_type=F32) + bo_ref[0, :]
  m = jnp.max(logits, axis=1, keepdims=True)
  lse = jnp.log(jnp.sum(jnp.exp(logits - m), axis=1, keepdims=True)) + m
  y_ref[...] = logits - lse


def kernel(x, edge_index, Wh, bh, W1_0, W1_1, W1_2, W1_3, Wo, bo):
  n, din = x.shape
  dh = Wh.shape[1]
  dout = Wo.shape[1]
  e = edge_index.shape[1]

  rows_per_tile = _ceil_div(n, NS * CSZ) * CSZ
  npad = rows_per_tile * NS
  chunks = _ceil_div(_ceil_div(e, NW), 2 * G * CSZ) * 2 * G
  epad = NW * chunks * CSZ

  pad = epad - e
  pad_iota = jnp.arange(pad, dtype=jnp.int32)
  src_r = jnp.concatenate(
      [edge_index[0], pad_iota % n]).reshape(NC, NS, chunks, CSZ)
  dst_r = jnp.concatenate(
      [edge_index[1], n + pad_iota % (npad - n)]).reshape(NC, NS, chunks, CSZ)
  xp = jnp.concatenate([x, jnp.zeros((npad - n, din), F32)])
  bh2 = bh.reshape(1, dh)
  bo2 = bo.reshape(1, dout)

  deg = _make_deg_kernel(chunks, npad)(dst_r)
  deg_t = deg.T

  R = 1024
  grid = (npad // R,)
  row_spec = pl.BlockSpec((R, din), lambda r: (r, 0))
  full_spec = pl.BlockSpec((din, dh), lambda r: (0, 0))

  x0, hp, dinv = pl.pallas_call(
      _prep_body,
      grid=grid,
      in_specs=[
          row_spec,
          full_spec,
          pl.BlockSpec((1, dh), lambda r: (0, 0)),
          pl.BlockSpec((R, 2), lambda r: (r, 0)),
      ],
      out_specs=[pl.BlockSpec((R, dh), lambda r: (r, 0))] * 3,
      out_shape=[jax.ShapeDtypeStruct((npad, dh), F32)] * 3,
  )(xp, Wh, bh2, deg_t)

  agg_call = _make_agg_kernel(chunks, npad, dh)
  layer_call = pl.pallas_call(
      _layer_body,
      grid=grid,
      in_specs=[
          pl.BlockSpec((NC, R, dh), lambda r: (0, r, 0)),
          pl.BlockSpec((R, dh), lambda r: (r, 0)),
          pl.BlockSpec((R, dh), lambda r: (r, 0)),
          pl.BlockSpec((R, dh), lambda r: (r, 0)),
          pl.BlockSpec((dh, dh), lambda r: (0, 0)),
      ],
      out_specs=[pl.BlockSpec((R, dh), lambda r: (r, 0))] * 2,
      out_shape=[jax.ShapeDtypeStruct((npad, dh), F32)] * 2,
  )

  h = x0
  for w1 in (W1_0, W1_1, W1_2):
    raw = agg_call(hp, src_r, dst_r)
    h, hp = layer_call(raw, h, x0, dinv, w1)

  raw = agg_call(hp, src_r, dst_r)
  y = pl.pallas_call(
      _final_body,
      grid=grid,
      in_specs=[
          pl.BlockSpec((NC, R, dh), lambda r: (0, r, 0)),
          pl.BlockSpec((R, dh), lambda r: (r, 0)),
          pl.BlockSpec((R, dh), lambda r: (r, 0)),
          pl.BlockSpec((R, dh), lambda r: (r, 0)),
          pl.BlockSpec((dh, dh), lambda r: (0, 0)),
          pl.BlockSpec((dh, dout), lambda r: (0, 0)),
          pl.BlockSpec((1, dout), lambda r: (0, 0)),
      ],
      out_specs=pl.BlockSpec((R, dout), lambda r: (r, 0)),
      out_shape=jax.ShapeDtypeStruct((npad, dout), F32),
  )(raw, h, x0, dinv, W1_3, Wo, bo2)

  return y[:n]


# speedup vs baseline: 4.3019x; 1.0391x over previous
import functools

import jax
import jax.numpy as jnp
from jax import lax
from jax.experimental import pallas as pl
from jax.experimental.pallas import tpu as pltpu
from jax.experimental.pallas import tpu_sc as plsc

F32 = jnp.float32
NC = 2
NS = 16
NW = NC * NS
CSZ = 128
G = 8
ALPHA = 0.1


def _ceil_div(a, b):
  return (a + b - 1) // b


def _make_deg_kernel(chunks, npad):
  mesh = plsc.VectorSubcoreMesh(core_axis_name="c", subcore_axis_name="s")
  rows_per_tile = npad // NS
  ncopy = rows_per_tile // CSZ

  @functools.partial(
      pl.kernel, mesh=mesh,
      out_type=jax.ShapeDtypeStruct((NC, npad), F32),
      scratch_types=[
          pltpu.VMEM((chunks, CSZ), jnp.int32),
          pltpu.VMEM((CSZ,), F32),
          pltpu.VMEM_SHARED((npad,), F32),
      ],
  )
  def deg_kernel(dst_hbm, out_hbm, dst_v, vec_v, acc_sh):
    c = lax.axis_index("c")
    s = lax.axis_index("s")
    pltpu.sync_copy(dst_hbm.at[c, s], dst_v)
    zeros16 = jnp.zeros((16,), F32)
    for j in range(CSZ // 16):
      vec_v[pl.ds(j * 16, 16)] = zeros16
    for k in range(ncopy):
      pltpu.sync_copy(vec_v, acc_sh.at[pl.ds(s * rows_per_tile + k * CSZ, CSZ)])
    plsc.subcore_barrier()
    ones16 = jnp.ones((16,), F32)
    for j in range(CSZ // 16):
      vec_v[pl.ds(j * 16, 16)] = ones16

    def body(k, carry):
      pltpu.sync_copy(vec_v, acc_sh.at[dst_v.at[k]], add=True)
      return carry

    lax.fori_loop(0, chunks, body, 0)
    plsc.subcore_barrier()
    pltpu.sync_copy(acc_sh.at[pl.ds(s * rows_per_tile, rows_per_tile)],
                    out_hbm.at[c, pl.ds(s * rows_per_tile, rows_per_tile)])

  return deg_kernel


def _make_agg_kernel(chunks, npad, d):
  mesh = plsc.VectorSubcoreMesh(core_axis_name="c", subcore_axis_name="s")
  rows_per_tile = npad // NS
  ncopy = rows_per_tile // CSZ
  ngroups = chunks // G

  @functools.partial(
      pl.kernel, mesh=mesh,
      out_type=jax.ShapeDtypeStruct((NC, npad, d), F32),
      scratch_types=[
          pltpu.VMEM((chunks, CSZ), jnp.int32),
          pltpu.VMEM((G, CSZ), jnp.int32),
          pltpu.VMEM((G, CSZ), jnp.int32),
          pltpu.VMEM((CSZ, d), F32),
          pltpu.VMEM((CSZ, d), F32),
          pltpu.SemaphoreType.DMA,
          pltpu.SemaphoreType.DMA,
          pltpu.SemaphoreType.DMA,
          pltpu.SemaphoreType.DMA,
          pltpu.SemaphoreType.DMA,
          pltpu.SemaphoreType.DMA,
          pltpu.VMEM_SHARED((npad, d), F32),
      ],
  )
  def agg_kernel(hp_hbm, src_hbm, dst_hbm, out_hbm,
                 src_v, dib0, dib1, rows0, rows1,
                 gsem0, gsem1, ssem0, ssem1, dsem0, dsem1, acc_sh):
    c = lax.axis_index("c")
    s = lax.axis_index("s")
    pltpu.sync_copy(src_hbm.at[c, s], src_v)

    zeros16 = jnp.zeros((16,), F32)

    def zbody(i, carry):
      r = i // (d // 16)
      col = (i % (d // 16)) * 16
      rows0[r, pl.ds(col, 16)] = zeros16
      return carry

    lax.fori_loop(0, CSZ * (d // 16), zbody, 0)
    for k in range(ncopy):
      pltpu.sync_copy(rows0, acc_sh.at[pl.ds(s * rows_per_tile + k * CSZ, CSZ)])
    plsc.subcore_barrier()

    rows = (rows0, rows1)
    gsem = (gsem0, gsem1)
    ssem = (ssem0, ssem1)
    dib = (dib0, dib1)
    dsem = (dsem0, dsem1)
    ngroups = chunks // G
    dg = [None, None]
    ds = [None, None]
    di = [None, None]
    dg[0] = pltpu.async_copy(hp_hbm.at[src_v.at[0]], rows0, gsem0)
    di[0] = pltpu.async_copy(dst_hbm.at[c, s, pl.ds(0, G)], dib0, dsem0)
    for k in range(chunks):
      p = k % 2
      g = k // G
      gp = g % 2
      if k >= 1:
        ds[1 - p].wait()
      if k % G == 0:
        if g + 1 < ngroups:
          di[1 - gp] = pltpu.async_copy(
              dst_hbm.at[c, s, pl.ds((g + 1) * G, G)], dib[1 - gp],
              dsem[1 - gp])
        di[gp].wait()
      if k + 1 < chunks:
        dg[1 - p] = pltpu.async_copy(
            hp_hbm.at[src_v.at[k + 1]], rows[1 - p], gsem[1 - p])
      dg[p].wait()
      ds[p] = pltpu.async_copy(
          rows[p], acc_sh.at[dib[gp].at[k % G]], ssem[p], add=True)
    ds[(chunks - 1) % 2].wait()
    plsc.subcore_barrier()
    for k in range(ncopy):
      r0 = s * rows_per_tile + k * CSZ
      pltpu.sync_copy(acc_sh.at[pl.ds(r0, CSZ)], out_hbm.at[c, pl.ds(r0, CSZ)])

  return agg_kernel


def _prep_body(x_ref, wh_ref, bh_ref, deg_ref, x0_ref, hp_ref, dinv_ref):
  deg = deg_ref[:, 0] + deg_ref[:, 1] + 1.0
  dinv = lax.rsqrt(deg)[:, None]
  x0 = jnp.dot(x_ref[...], wh_ref[...], preferred_element_type=F32) + bh_ref[0, :]
  x0_ref[...] = x0
  hp_ref[...] = x0 * dinv
  dinv_ref[...] = jnp.broadcast_to(dinv, x0.shape)


def _layer_body(raw_ref, h_ref, x0_ref, dinv_ref, w_ref, h1_ref, hp1_ref):
  dinv = dinv_ref[...]
  raw = raw_ref[0] + raw_ref[1]
  agg = dinv * (raw + dinv * h_ref[...])
  xmix = (1.0 - ALPHA) * agg + ALPHA * x0_ref[...]
  out = jnp.dot(xmix, w_ref[...], preferred_element_type=F32)
  h1 = jnp.maximum(out, 0.0)
  h1_ref[...] = h1
  hp1_ref[...] = dinv * h1


def _final_body(raw_ref, h_ref, x0_ref, dinv_ref, w_ref, wo_ref, bo_ref, y_ref):
  dinv = dinv_ref[...]
  raw = raw_ref[0] + raw_ref[1]
  agg = dinv * (raw + dinv * h_ref[...])
  xmix = (1.0 - ALPHA) * agg + ALPHA * x0_ref[...]
  out = jnp.dot(xmix, w_ref[...], preferred_element_type=F32)
  logits = jnp.dot(out, wo_ref[...], preferred_element_type=F32) + bo_ref[0, :]
  m = jnp.max(logits, axis=1, keepdims=True)
  lse = jnp.log(jnp.sum(jnp.exp(logits - m), axis=1, keepdims=True)) + m
  y_ref[...] = logits - lse


def kernel(x, edge_index, Wh, bh, W1_0, W1_1, W1_2, W1_3, Wo, bo):
  n, din = x.shape
  dh = Wh.shape[1]
  dout = Wo.shape[1]
  e = edge_index.shape[1]

  rows_per_tile = _ceil_div(n, NS * CSZ) * CSZ
  npad = rows_per_tile * NS
  chunks = _ceil_div(_ceil_div(e, NW), 2 * G * CSZ) * 2 * G
  epad = NW * chunks * CSZ

  pad = epad - e
  pad_iota = jnp.arange(pad, dtype=jnp.int32)
  src_r = jnp.concatenate(
      [edge_index[0], pad_iota % n]).reshape(NC, NS, chunks, CSZ)
  dst_r = jnp.concatenate(
      [edge_index[1], n + pad_iota % (npad - n)]).reshape(NC, NS, chunks, CSZ)
  xp = jnp.concatenate([x, jnp.zeros((npad - n, din), F32)])
  bh2 = bh.reshape(1, dh)
  bo2 = bo.reshape(1, dout)

  deg = _make_deg_kernel(chunks, npad)(dst_r)
  deg_t = deg.T

  R = 1024
  grid = (npad // R,)
  row_spec = pl.BlockSpec((R, din), lambda r: (r, 0))
  full_spec = pl.BlockSpec((din, dh), lambda r: (0, 0))

  x0, hp, dinv = pl.pallas_call(
      _prep_body,
      grid=grid,
      in_specs=[
          row_spec,
          full_spec,
          pl.BlockSpec((1, dh), lambda r: (0, 0)),
          pl.BlockSpec((R, 2), lambda r: (r, 0)),
      ],
      out_specs=[pl.BlockSpec((R, dh), lambda r: (r, 0))] * 3,
      out_shape=[jax.ShapeDtypeStruct((npad, dh), F32)] * 3,
  )(xp, Wh, bh2, deg_t)

  agg_call = _make_agg_kernel(chunks, npad, dh)
  layer_call = pl.pallas_call(
      _layer_body,
      grid=grid,
      in_specs=[
          pl.BlockSpec((NC, R, dh), lambda r: (0, r, 0)),
          pl.BlockSpec((R, dh), lambda r: (r, 0)),
          pl.BlockSpec((R, dh), lambda r: (r, 0)),
          pl.BlockSpec((R, dh), lambda r: (r, 0)),
          pl.BlockSpec((dh, dh), lambda r: (0, 0)),
      ],
      out_specs=[pl.BlockSpec((R, dh), lambda r: (r, 0))] * 2,
      out_shape=[jax.ShapeDtypeStruct((npad, dh), F32)] * 2,
  )

  h = x0
  for w1 in (W1_0, W1_1, W1_2):
    raw = agg_call(hp, src_r, dst_r)
    h, hp = layer_call(raw, h, x0, dinv, w1)

  raw = agg_call(hp, src_r, dst_r)
  y = pl.pallas_call(
      _final_body,
      grid=grid,
      in_specs=[
          pl.BlockSpec((NC, R, dh), lambda r: (0, r, 0)),
          pl.BlockSpec((R, dh), lambda r: (r, 0)),
          pl.BlockSpec((R, dh), lambda r: (r, 0)),
          pl.BlockSpec((R, dh), lambda r: (r, 0)),
          pl.BlockSpec((dh, dh), lambda r: (0, 0)),
          pl.BlockSpec((dh, dout), lambda r: (0, 0)),
          pl.BlockSpec((1, dout), lambda r: (0, 0)),
      ],
      out_specs=pl.BlockSpec((R, dout), lambda r: (r, 0)),
      out_shape=jax.ShapeDtypeStruct((npad, dout), F32),
  )(raw, h, x0, dinv, W1_3, Wo, bo2)

  return y[:n]


# speedup vs baseline: 4.3126x; 1.0025x over previous
import functools

import jax
import jax.numpy as jnp
from jax import lax
from jax.experimental import pallas as pl
from jax.experimental.pallas import tpu as pltpu
from jax.experimental.pallas import tpu_sc as plsc

F32 = jnp.float32
NC = 2
NS = 16
NW = NC * NS
CSZ = 128
G = 8
ALPHA = 0.1


def _ceil_div(a, b):
  return (a + b - 1) // b


def _make_deg_kernel(chunks, npad):
  mesh = plsc.VectorSubcoreMesh(core_axis_name="c", subcore_axis_name="s")
  rows_per_tile = npad // NS
  ncopy = rows_per_tile // CSZ

  @functools.partial(
      pl.kernel, mesh=mesh,
      out_type=jax.ShapeDtypeStruct((NC, npad), F32),
      scratch_types=[
          pltpu.VMEM((chunks, CSZ), jnp.int32),
          pltpu.VMEM((CSZ,), F32),
          pltpu.VMEM_SHARED((npad,), F32),
      ],
  )
  def deg_kernel(dst_hbm, out_hbm, dst_v, vec_v, acc_sh):
    c = lax.axis_index("c")
    s = lax.axis_index("s")
    pltpu.sync_copy(dst_hbm.at[c, s], dst_v)
    zeros16 = jnp.zeros((16,), F32)
    for j in range(CSZ // 16):
      vec_v[pl.ds(j * 16, 16)] = zeros16
    for k in range(ncopy):
      pltpu.sync_copy(vec_v, acc_sh.at[pl.ds(s * rows_per_tile + k * CSZ, CSZ)])
    plsc.subcore_barrier()
    ones16 = jnp.ones((16,), F32)
    for j in range(CSZ // 16):
      vec_v[pl.ds(j * 16, 16)] = ones16

    def body(k, carry):
      pltpu.sync_copy(vec_v, acc_sh.at[dst_v.at[k]], add=True)
      return carry

    lax.fori_loop(0, chunks, body, 0)
    plsc.subcore_barrier()
    pltpu.sync_copy(acc_sh.at[pl.ds(s * rows_per_tile, rows_per_tile)],
                    out_hbm.at[c, pl.ds(s * rows_per_tile, rows_per_tile)])

  return deg_kernel


def _make_agg_kernel(chunks, npad, d):
  mesh = plsc.VectorSubcoreMesh(core_axis_name="c", subcore_axis_name="s")
  rows_per_tile = npad // NS
  ncopy = rows_per_tile // CSZ

  @functools.partial(
      pl.kernel, mesh=mesh,
      out_type=jax.ShapeDtypeStruct((NC, npad, d), F32),
      scratch_types=[
          pltpu.VMEM((chunks, CSZ), jnp.int32),
          pltpu.VMEM((G, CSZ), jnp.int32),
          pltpu.VMEM((G, CSZ), jnp.int32),
          pltpu.VMEM((CSZ, d), F32),
          pltpu.VMEM((CSZ, d), F32),
          pltpu.SemaphoreType.DMA,
          pltpu.SemaphoreType.DMA,
          pltpu.SemaphoreType.DMA,
          pltpu.SemaphoreType.DMA,
          pltpu.SemaphoreType.DMA,
          pltpu.SemaphoreType.DMA,
          pltpu.VMEM_SHARED((npad, d), F32),
      ],
  )
  def agg_kernel(hp_hbm, src_hbm, dst_hbm, out_hbm,
                 src_v, dib0, dib1, rows0, rows1,
                 gsem0, gsem1, ssem0, ssem1, dsem0, dsem1, acc_sh):
    c = lax.axis_index("c")
    s = lax.axis_index("s")
    pltpu.sync_copy(src_hbm.at[c, s], src_v)

    zeros16 = jnp.zeros((16,), F32)

    def zbody(i, carry):
      r = i // (d // 16)
      col = (i % (d // 16)) * 16
      rows0[r, pl.ds(col, 16)] = zeros16
      return carry

    lax.fori_loop(0, CSZ * (d // 16), zbody, 0)
    for k in range(ncopy):
      pltpu.sync_copy(rows0, acc_sh.at[pl.ds(s * rows_per_tile + k * CSZ, CSZ)])
    plsc.subcore_barrier()

    rows = (rows0, rows1)
    gsem = (gsem0, gsem1)
    ssem = (ssem0, ssem1)
    dib = (dib0, dib1)
    dsem = (dsem0, dsem1)
    ngroups = chunks // G
    dg = [None, None]
    ds = [None, None]
    di = [None, None]
    dg[0] = pltpu.async_copy(hp_hbm.at[src_v.at[0]], rows0, gsem0)
    di[0] = pltpu.async_copy(dst_hbm.at[c, s, pl.ds(0, G)], dib0, dsem0)
    for k in range(chunks):
      p = k % 2
      g = k // G
      gp = g % 2
      if k >= 1:
        ds[1 - p].wait()
      if k % G == 0:
        if g + 1 < ngroups:
          di[1 - gp] = pltpu.async_copy(
              dst_hbm.at[c, s, pl.ds((g + 1) * G, G)], dib[1 - gp],
              dsem[1 - gp])
        di[gp].wait()
      if k + 1 < chunks:
        dg[1 - p] = pltpu.async_copy(
            hp_hbm.at[src_v.at[k + 1]], rows[1 - p], gsem[1 - p])
      dg[p].wait()
      ds[p] = pltpu.async_copy(
          rows[p], acc_sh.at[dib[gp].at[k % G]], ssem[p], add=True)
    ds[(chunks - 1) % 2].wait()
    plsc.subcore_barrier()
    for k in range(ncopy):
      r0 = s * rows_per_tile + k * CSZ
      pltpu.sync_copy(acc_sh.at[pl.ds(r0, CSZ)], out_hbm.at[c, pl.ds(r0, CSZ)])

  return agg_kernel


def _prep_body(x_ref, wh_ref, bh_ref, deg_ref, x0_ref, hp_ref, dinv_ref):
  deg = deg_ref[:, 0] + deg_ref[:, 1] + 1.0
  dinv = lax.rsqrt(deg)[:, None]
  x0 = jnp.dot(x_ref[...], wh_ref[...], preferred_element_type=F32) + bh_ref[0, :]
  x0_ref[...] = x0
  hp_ref[...] = x0 * dinv
  dinv_ref[...] = jnp.broadcast_to(dinv, x0.shape)


def _layer_body(raw_ref, h_ref, x0_ref, dinv_ref, w_ref, h1_ref, hp1_ref):
  dinv = dinv_ref[...]
  raw = raw_ref[0] + raw_ref[1]
  agg = dinv * (raw + dinv * h_ref[...])
  xmix = (1.0 - ALPHA) * agg + ALPHA * x0_ref[...]
  out = jnp.dot(xmix, w_ref[...], preferred_element_type=F32)
  h1 = jnp.maximum(out, 0.0)
  h1_ref[...] = h1
  hp1_ref[...] = dinv * h1


def _final_body(raw_ref, h_ref, x0_ref, dinv_ref, w_ref, wo_ref, bo_ref, y_ref):
  dinv = dinv_ref[...]
  raw = raw_ref[0] + raw_ref[1]
  agg = dinv * (raw + dinv * h_ref[...])
  xmix = (1.0 - ALPHA) * agg + ALPHA * x0_ref[...]
  out = jnp.dot(xmix, w_ref[...], preferred_element_type=F32)
  logits = jnp.dot(out, wo_ref[...], preferred_element_type=F32) + bo_ref[0, :]
  m = jnp.max(logits, axis=1, keepdims=True)
  lse = jnp.log(jnp.sum(jnp.exp(logits - m), axis=1, keepdims=True)) + m
  y_ref[...] = logits - lse


def kernel(x, edge_index, Wh, bh, W1_0, W1_1, W1_2, W1_3, Wo, bo):
  n, din = x.shape
  dh = Wh.shape[1]
  dout = Wo.shape[1]
  e = edge_index.shape[1]

  rows_per_tile = _ceil_div(n, NS * CSZ) * CSZ
  npad = rows_per_tile * NS
  chunks = _ceil_div(_ceil_div(e, NW), 2 * G * CSZ) * 2 * G
  epad = NW * chunks * CSZ

  pad = epad - e
  pad_iota = jnp.arange(pad, dtype=jnp.int32)
  src_r = jnp.concatenate(
      [edge_index[0], pad_iota % n]).reshape(NC, NS, chunks, CSZ)
  dst_r = jnp.concatenate(
      [edge_index[1], n + pad_iota % (npad - n)]).reshape(NC, NS, chunks, CSZ)
  xp = jnp.concatenate([x, jnp.zeros((npad - n, din), F32)])
  bh2 = bh.reshape(1, dh)
  bo2 = bo.reshape(1, dout)

  deg = _make_deg_kernel(chunks, npad)(dst_r)
  deg_t = deg.T

  R = 1024
  grid = (npad // R,)
  row_spec = pl.BlockSpec((R, din), lambda r: (r, 0))
  full_spec = pl.BlockSpec((din, dh), lambda r: (0, 0))

  x0, hp, dinv = pl.pallas_call(
      _prep_body,
      grid=grid,
      in_specs=[
          row_spec,
          full_spec,
          pl.BlockSpec((1, dh), lambda r: (0, 0)),
          pl.BlockSpec((R, 2), lambda r: (r, 0)),
      ],
      out_specs=[pl.BlockSpec((R, dh), lambda r: (r, 0))] * 3,
      out_shape=[jax.ShapeDtypeStruct((npad, dh), F32)] * 3,
  )(xp, Wh, bh2, deg_t)

  agg_call = _make_agg_kernel(chunks, npad, dh)
  layer_call = pl.pallas_call(
      _layer_body,
      grid=grid,
      in_specs=[
          pl.BlockSpec((NC, R, dh), lambda r: (0, r, 0)),
          pl.BlockSpec((R, dh), lambda r: (r, 0)),
          pl.BlockSpec((R, dh), lambda r: (r, 0)),
          pl.BlockSpec((R, dh), lambda r: (r, 0)),
          pl.BlockSpec((dh, dh), lambda r: (0, 0)),
      ],
      out_specs=[pl.BlockSpec((R, dh), lambda r: (r, 0))] * 2,
      out_shape=[jax.ShapeDtypeStruct((npad, dh), F32)] * 2,
  )

  h = x0
  for w1 in (W1_0, W1_1, W1_2):
    raw = agg_call(hp, src_r, dst_r)
    h, hp = layer_call(raw, h, x0, dinv, w1)

  raw = agg_call(hp, src_r, dst_r)
  y = pl.pallas_call(
      _final_body,
      grid=grid,
      in_specs=[
          pl.BlockSpec((NC, R, dh), lambda r: (0, r, 0)),
          pl.BlockSpec((R, dh), lambda r: (r, 0)),
          pl.BlockSpec((R, dh), lambda r: (r, 0)),
          pl.BlockSpec((R, dh), lambda r: (r, 0)),
          pl.BlockSpec((dh, dh), lambda r: (0, 0)),
          pl.BlockSpec((dh, dout), lambda r: (0, 0)),
          pl.BlockSpec((1, dout), lambda r: (0, 0)),
      ],
      out_specs=pl.BlockSpec((R, dout), lambda r: (r, 0)),
      out_shape=jax.ShapeDtypeStruct((npad, dout), F32),
  )(raw, h, x0, dinv, W1_3, Wo, bo2)

  return y[:n]


# speedup vs baseline: 4.3152x; 1.0006x over previous
import functools

import jax
import jax.numpy as jnp
from jax import lax
from jax.experimental import pallas as pl
from jax.experimental.pallas import tpu as pltpu
from jax.experimental.pallas import tpu_sc as plsc

F32 = jnp.float32
NC = 2
NS = 16
NW = NC * NS
CSZ = 128
G = 8
ALPHA = 0.1


def _ceil_div(a, b):
  return (a + b - 1) // b


def _make_deg_kernel(chunks, npad):
  mesh = plsc.VectorSubcoreMesh(core_axis_name="c", subcore_axis_name="s")
  rows_per_tile = npad // NS
  ncopy = rows_per_tile // CSZ

  @functools.partial(
      pl.kernel, mesh=mesh,
      out_type=jax.ShapeDtypeStruct((NC, npad), F32),
      scratch_types=[
          pltpu.VMEM((chunks, CSZ), jnp.int32),
          pltpu.VMEM((CSZ,), F32),
          pltpu.VMEM_SHARED((npad,), F32),
      ],
  )
  def deg_kernel(dst_hbm, out_hbm, dst_v, vec_v, acc_sh):
    c = lax.axis_index("c")
    s = lax.axis_index("s")
    pltpu.sync_copy(dst_hbm.at[c, s], dst_v)
    zeros16 = jnp.zeros((16,), F32)
    for j in range(CSZ // 16):
      vec_v[pl.ds(j * 16, 16)] = zeros16
    for k in range(ncopy):
      pltpu.sync_copy(vec_v, acc_sh.at[pl.ds(s * rows_per_tile + k * CSZ, CSZ)])
    plsc.subcore_barrier()
    ones16 = jnp.ones((16,), F32)
    for j in range(CSZ // 16):
      vec_v[pl.ds(j * 16, 16)] = ones16

    def body(k, carry):
      pltpu.sync_copy(vec_v, acc_sh.at[dst_v.at[k]], add=True)
      return carry

    lax.fori_loop(0, chunks, body, 0)
    plsc.subcore_barrier()
    pltpu.sync_copy(acc_sh.at[pl.ds(s * rows_per_tile, rows_per_tile)],
                    out_hbm.at[c, pl.ds(s * rows_per_tile, rows_per_tile)])

  return deg_kernel


def _make_agg_kernel(chunks, npad, d):
  mesh = plsc.VectorSubcoreMesh(core_axis_name="c", subcore_axis_name="s")
  rows_per_tile = npad // NS
  ncopy = rows_per_tile // CSZ

  @functools.partial(
      pl.kernel, mesh=mesh,
      out_type=jax.ShapeDtypeStruct((NC, npad, d), F32),
      scratch_types=[
          pltpu.VMEM((chunks, CSZ), jnp.int32),
          pltpu.VMEM((G, CSZ), jnp.int32),
          pltpu.VMEM((G, CSZ), jnp.int32),
          pltpu.VMEM((CSZ, d), F32),
          pltpu.VMEM((CSZ, d), F32),
          pltpu.SemaphoreType.DMA,
          pltpu.SemaphoreType.DMA,
          pltpu.SemaphoreType.DMA,
          pltpu.SemaphoreType.DMA,
          pltpu.SemaphoreType.DMA,
          pltpu.SemaphoreType.DMA,
          pltpu.VMEM_SHARED((npad, d), F32),
      ],
  )
  def agg_kernel(hp_hbm, src_hbm, dst_hbm, out_hbm,
                 src_v, dib0, dib1, rows0, rows1,
                 gsem0, gsem1, ssem0, ssem1, dsem0, dsem1, acc_sh):
    c = lax.axis_index("c")
    s = lax.axis_index("s")
    pltpu.sync_copy(src_hbm.at[c, s], src_v)

    zeros16 = jnp.zeros((16,), F32)

    def zbody(i, carry):
      r = i // (d // 16)
      col = (i % (d // 16)) * 16
      rows0[r, pl.ds(col, 16)] = zeros16
      return carry

    lax.fori_loop(0, CSZ * (d // 16), zbody, 0)
    for k in range(ncopy):
      pltpu.sync_copy(rows0, acc_sh.at[pl.ds(s * rows_per_tile + k * CSZ, CSZ)])
    plsc.subcore_barrier()

    rows = (rows0, rows1)
    gsem = (gsem0, gsem1)
    ssem = (ssem0, ssem1)
    dib = (dib0, dib1)
    dsem = (dsem0, dsem1)
    ngroups = chunks // G
    dg = [None, None]
    ds = [None, None]
    di = [None, None]
    dg[0] = pltpu.async_copy(hp_hbm.at[src_v.at[0]], rows0, gsem0)
    di[0] = pltpu.async_copy(dst_hbm.at[c, s, pl.ds(0, G)], dib0, dsem0)
    for k in range(chunks):
      p = k % 2
      g = k // G
      gp = g % 2
      if k >= 1:
        ds[1 - p].wait()
      if k % G == 0:
        if g + 1 < ngroups:
          di[1 - gp] = pltpu.async_copy(
              dst_hbm.at[c, s, pl.ds((g + 1) * G, G)], dib[1 - gp],
              dsem[1 - gp])
        di[gp].wait()
      if k + 1 < chunks:
        dg[1 - p] = pltpu.async_copy(
            hp_hbm.at[src_v.at[k + 1]], rows[1 - p], gsem[1 - p])
      dg[p].wait()
      ds[p] = pltpu.async_copy(
          rows[p], acc_sh.at[dib[gp].at[k % G]], ssem[p], add=True)
    ds[(chunks - 1) % 2].wait()
    plsc.subcore_barrier()
    for k in range(ncopy):
      r0 = s * rows_per_tile + k * CSZ
      pltpu.sync_copy(acc_sh.at[pl.ds(r0, CSZ)], out_hbm.at[c, pl.ds(r0, CSZ)])

  return agg_kernel


def _mm_body(x_ref, wh_ref, bh_ref, x0_ref):
  x0_ref[...] = (
      jnp.dot(x_ref[...], wh_ref[...], preferred_element_type=F32) + bh_ref[0, :])


def _scale_body(x0_ref, deg_ref, hp_ref, dinv_ref):
  deg = deg_ref[:, 0] + deg_ref[:, 1] + 1.0
  dinv = lax.rsqrt(deg)[:, None]
  x0 = x0_ref[...]
  hp_ref[...] = x0 * dinv
  dinv_ref[...] = jnp.broadcast_to(dinv, x0.shape)


def _layer_body(raw_ref, h_ref, x0_ref, dinv_ref, w_ref, h1_ref, hp1_ref):
  dinv = dinv_ref[...]
  raw = raw_ref[0] + raw_ref[1]
  agg = dinv * (raw + dinv * h_ref[...])
  xmix = (1.0 - ALPHA) * agg + ALPHA * x0_ref[...]
  out = jnp.dot(xmix, w_ref[...], preferred_element_type=F32)
  h1 = jnp.maximum(out, 0.0)
  h1_ref[...] = h1
  hp1_ref[...] = dinv * h1


def _final_body(raw_ref, h_ref, x0_ref, dinv_ref, w_ref, wo_ref, bo_ref, y_ref):
  dinv = dinv_ref[...]
  raw = raw_ref[0] + raw_ref[1]
  agg = dinv * (raw + dinv * h_ref[...])
  xmix = (1.0 - ALPHA) * agg + ALPHA * x0_ref[...]
  out = jnp.dot(xmix, w_ref[...], preferred_element_type=F32)
  logits = jnp.dot(out, wo_ref[...], preferred_element_type=F32) + bo_ref[0, :]
  m = jnp.max(logits, axis=1, keepdims=True)
  lse = jnp.log(jnp.sum(jnp.exp(logits - m), axis=1, keepdims=True)) + m
  y_ref[...] = logits - lse


def kernel(x, edge_index, Wh, bh, W1_0, W1_1, W1_2, W1_3, Wo, bo):
  n, din = x.shape
  dh = Wh.shape[1]
  dout = Wo.shape[1]
  e = edge_index.shape[1]

  rows_per_tile = _ceil_div(n, NS * CSZ) * CSZ
  npad = rows_per_tile * NS
  chunks = _ceil_div(_ceil_div(e, NW), 2 * G * CSZ) * 2 * G
  epad = NW * chunks * CSZ

  pad = epad - e
  pad_iota = jnp.arange(pad, dtype=jnp.int32)
  src_r = jnp.concatenate(
      [edge_index[0], pad_iota % n]).reshape(NC, NS, chunks, CSZ)
  dst_r = jnp.concatenate(
      [edge_index[1], n + pad_iota % (npad - n)]).reshape(NC, NS, chunks, CSZ)
  xp = jnp.concatenate([x, jnp.zeros((npad - n, din), F32)])
  bh2 = bh.reshape(1, dh)
  bo2 = bo.reshape(1, dout)

  deg = _make_deg_kernel(chunks, npad)(dst_r)

  R = 1024
  grid = (npad // R,)
  row_spec = pl.BlockSpec((R, din), lambda r: (r, 0))
  full_spec = pl.BlockSpec((din, dh), lambda r: (0, 0))

  x0 = pl.pallas_call(
      _mm_body,
      grid=grid,
      in_specs=[
          row_spec,
          full_spec,
          pl.BlockSpec((1, dh), lambda r: (0, 0)),
      ],
      out_specs=pl.BlockSpec((R, dh), lambda r: (r, 0)),
      out_shape=jax.ShapeDtypeStruct((npad, dh), F32),
  )(xp, Wh, bh2)

  deg_t = deg.T
  hp, dinv = pl.pallas_call(
      _scale_body,
      grid=grid,
      in_specs=[
          pl.BlockSpec((R, dh), lambda r: (r, 0)),
          pl.BlockSpec((R, 2), lambda r: (r, 0)),
      ],
      out_specs=[pl.BlockSpec((R, dh), lambda r: (r, 0))] * 2,
      out_shape=[jax.ShapeDtypeStruct((npad, dh), F32)] * 2,
  )(x0, deg_t)

  agg_call = _make_agg_kernel(chunks, npad, dh)
  layer_call = pl.pallas_call(
      _layer_body,
      grid=grid,
      in_specs=[
          pl.BlockSpec((NC, R, dh), lambda r: (0, r, 0)),
          pl.BlockSpec((R, dh), lambda r: (r, 0)),
          pl.BlockSpec((R, dh), lambda r: (r, 0)),
          pl.BlockSpec((R, dh), lambda r: (r, 0)),
          pl.BlockSpec((dh, dh), lambda r: (0, 0)),
      ],
      out_specs=[pl.BlockSpec((R, dh), lambda r: (r, 0))] * 2,
      out_shape=[jax.ShapeDtypeStruct((npad, dh), F32)] * 2,
  )

  h = x0
  for w1 in (W1_0, W1_1, W1_2):
    raw = agg_call(hp, src_r, dst_r)
    h, hp = layer_call(raw, h, x0, dinv, w1)

  raw = agg_call(hp, src_r, dst_r)
  y = pl.pallas_call(
      _final_body,
      grid=grid,
      in_specs=[
          pl.BlockSpec((NC, R, dh), lambda r: (0, r, 0)),
          pl.BlockSpec((R, dh), lambda r: (r, 0)),
          pl.BlockSpec((R, dh), lambda r: (r, 0)),
          pl.BlockSpec((R, dh), lambda r: (r, 0)),
          pl.BlockSpec((dh, dh), lambda r: (0, 0)),
          pl.BlockSpec((dh, dout), lambda r: (0, 0)),
          pl.BlockSpec((1, dout), lambda r: (0, 0)),
      ],
      out_specs=pl.BlockSpec((R, dout), lambda r: (r, 0)),
      out_shape=jax.ShapeDtypeStruct((npad, dout), F32),
  )(raw, h, x0, dinv, W1_3, Wo, bo2)

  return y[:n]


# speedup vs baseline: 4.3272x; 1.0028x over previous
import functools

import jax
import jax.numpy as jnp
from jax import lax
from jax.experimental import pallas as pl
from jax.experimental.pallas import tpu as pltpu
from jax.experimental.pallas import tpu_sc as plsc

F32 = jnp.float32
NC = 2
NS = 16
NW = NC * NS
CSZ = 128
G = 8
ALPHA = 0.1


def _ceil_div(a, b):
  return (a + b - 1) // b


def _make_deg_kernel(chunks, npad):
  mesh = plsc.VectorSubcoreMesh(core_axis_name="c", subcore_axis_name="s")
  rows_per_tile = npad // NS
  ncopy = rows_per_tile // CSZ

  @functools.partial(
      pl.kernel, mesh=mesh,
      out_type=jax.ShapeDtypeStruct((NC, npad), F32),
      scratch_types=[
          pltpu.VMEM((chunks, CSZ), jnp.int32),
          pltpu.VMEM((CSZ,), F32),
          pltpu.VMEM_SHARED((npad,), F32),
      ],
  )
  def deg_kernel(dst_hbm, out_hbm, dst_v, vec_v, acc_sh):
    c = lax.axis_index("c")
    s = lax.axis_index("s")
    pltpu.sync_copy(dst_hbm.at[c, s], dst_v)
    zeros16 = jnp.zeros((16,), F32)
    for j in range(CSZ // 16):
      vec_v[pl.ds(j * 16, 16)] = zeros16
    for k in range(ncopy):
      pltpu.sync_copy(vec_v, acc_sh.at[pl.ds(s * rows_per_tile + k * CSZ, CSZ)])
    plsc.subcore_barrier()
    ones16 = jnp.ones((16,), F32)
    for j in range(CSZ // 16):
      vec_v[pl.ds(j * 16, 16)] = ones16

    def body(k, carry):
      pltpu.sync_copy(vec_v, acc_sh.at[dst_v.at[k]], add=True)
      return carry

    lax.fori_loop(0, chunks, body, 0)
    plsc.subcore_barrier()
    pltpu.sync_copy(acc_sh.at[pl.ds(s * rows_per_tile, rows_per_tile)],
                    out_hbm.at[c, pl.ds(s * rows_per_tile, rows_per_tile)])

  return deg_kernel


def _make_agg_kernel(chunks, npad, d):
  mesh = plsc.VectorSubcoreMesh(core_axis_name="c", subcore_axis_name="s")
  rows_per_tile = npad // NS
  ncopy = rows_per_tile // CSZ

  @functools.partial(
      pl.kernel, mesh=mesh,
      out_type=jax.ShapeDtypeStruct((NC, npad, d), F32),
      scratch_types=[
          pltpu.VMEM((chunks, CSZ), jnp.int32),
          pltpu.VMEM((G, CSZ), jnp.int32),
          pltpu.VMEM((G, CSZ), jnp.int32),
          pltpu.VMEM((CSZ, d), F32),
          pltpu.VMEM((CSZ, d), F32),
          pltpu.SemaphoreType.DMA,
          pltpu.SemaphoreType.DMA,
          pltpu.SemaphoreType.DMA,
          pltpu.SemaphoreType.DMA,
          pltpu.SemaphoreType.DMA,
          pltpu.SemaphoreType.DMA,
          pltpu.VMEM_SHARED((npad, d), F32),
      ],
  )
  def agg_kernel(hp_hbm, src_hbm, dst_hbm, out_hbm,
                 src_v, dib0, dib1, rows0, rows1,
                 gsem0, gsem1, ssem0, ssem1, dsem0, dsem1, acc_sh):
    c = lax.axis_index("c")
    s = lax.axis_index("s")
    dsrc = pltpu.async_copy(src_hbm.at[c, s], src_v, gsem0)

    zeros16 = jnp.zeros((16,), F32)

    def zbody(i, carry):
      r = i // (d // 16)
      col = (i % (d // 16)) * 16
      rows0[r, pl.ds(col, 16)] = zeros16
      return carry

    lax.fori_loop(0, CSZ * (d // 16), zbody, 0)
    zcopies = [
        pltpu.async_copy(
            rows0, acc_sh.at[pl.ds(s * rows_per_tile + k * CSZ, CSZ)], ssem0)
        for k in range(ncopy)
    ]
    for dz in zcopies:
      dz.wait()
    dsrc.wait()
    plsc.subcore_barrier()

    rows = (rows0, rows1)
    gsem = (gsem0, gsem1)
    ssem = (ssem0, ssem1)
    dib = (dib0, dib1)
    dsem = (dsem0, dsem1)
    ngroups = chunks // G
    dg = [None, None]
    ds = [None, None]
    di = [None, None]
    dg[0] = pltpu.async_copy(hp_hbm.at[src_v.at[0]], rows0, gsem0)
    di[0] = pltpu.async_copy(dst_hbm.at[c, s, pl.ds(0, G)], dib0, dsem0)
    for k in range(chunks):
      p = k % 2
      g = k // G
      gp = g % 2
      if k >= 1:
        ds[1 - p].wait()
      if k % G == 0:
        if g + 1 < ngroups:
          di[1 - gp] = pltpu.async_copy(
              dst_hbm.at[c, s, pl.ds((g + 1) * G, G)], dib[1 - gp],
              dsem[1 - gp])
        di[gp].wait()
      if k + 1 < chunks:
        dg[1 - p] = pltpu.async_copy(
            hp_hbm.at[src_v.at[k + 1]], rows[1 - p], gsem[1 - p])
      dg[p].wait()
      ds[p] = pltpu.async_copy(
          rows[p], acc_sh.at[dib[gp].at[k % G]], ssem[p], add=True)
    ds[(chunks - 1) % 2].wait()
    plsc.subcore_barrier()
    wcopies = []
    for k in range(ncopy):
      r0 = s * rows_per_tile + k * CSZ
      wcopies.append(pltpu.async_copy(
          acc_sh.at[pl.ds(r0, CSZ)], out_hbm.at[c, pl.ds(r0, CSZ)], ssem0))
    for dw in wcopies:
      dw.wait()

  return agg_kernel


def _mm_body(x_ref, wh_ref, bh_ref, x0_ref):
  x0_ref[...] = (
      jnp.dot(x_ref[...], wh_ref[...], preferred_element_type=F32) + bh_ref[0, :])


def _scale_body(x0_ref, deg_ref, hp_ref, dinv_ref):
  deg = deg_ref[:, 0] + deg_ref[:, 1] + 1.0
  dinv = lax.rsqrt(deg)[:, None]
  x0 = x0_ref[...]
  hp_ref[...] = x0 * dinv
  dinv_ref[...] = jnp.broadcast_to(dinv, x0.shape)


def _layer_body(raw_ref, h_ref, x0_ref, dinv_ref, w_ref, h1_ref, hp1_ref):
  dinv = dinv_ref[...]
  raw = raw_ref[0] + raw_ref[1]
  agg = dinv * (raw + dinv * h_ref[...])
  xmix = (1.0 - ALPHA) * agg + ALPHA * x0_ref[...]
  out = jnp.dot(xmix, w_ref[...], preferred_element_type=F32)
  h1 = jnp.maximum(out, 0.0)
  h1_ref[...] = h1
  hp1_ref[...] = dinv * h1


def _final_body(raw_ref, h_ref, x0_ref, dinv_ref, w_ref, wo_ref, bo_ref, y_ref):
  dinv = dinv_ref[...]
  raw = raw_ref[0] + raw_ref[1]
  agg = dinv * (raw + dinv * h_ref[...])
  xmix = (1.0 - ALPHA) * agg + ALPHA * x0_ref[...]
  out = jnp.dot(xmix, w_ref[...], preferred_element_type=F32)
  logits = jnp.dot(out, wo_ref[...], preferred_element_type=F32) + bo_ref[0, :]
  m = jnp.max(logits, axis=1, keepdims=True)
  lse = jnp.log(jnp.sum(jnp.exp(logits - m), axis=1, keepdims=True)) + m
  y_ref[...] = logits - lse


def kernel(x, edge_index, Wh, bh, W1_0, W1_1, W1_2, W1_3, Wo, bo):
  n, din = x.shape
  dh = Wh.shape[1]
  dout = Wo.shape[1]
  e = edge_index.shape[1]

  rows_per_tile = _ceil_div(n, NS * CSZ) * CSZ
  npad = rows_per_tile * NS
  chunks = _ceil_div(_ceil_div(e, NW), 2 * G * CSZ) * 2 * G
  epad = NW * chunks * CSZ

  pad = epad - e
  pad_iota = jnp.arange(pad, dtype=jnp.int32)
  src_r = jnp.concatenate(
      [edge_index[0], pad_iota % n]).reshape(NC, NS, chunks, CSZ)
  dst_r = jnp.concatenate(
      [edge_index[1], n + pad_iota % (npad - n)]).reshape(NC, NS, chunks, CSZ)
  xp = jnp.concatenate([x, jnp.zeros((npad - n, din), F32)])
  bh2 = bh.reshape(1, dh)
  bo2 = bo.reshape(1, dout)

  deg = _make_deg_kernel(chunks, npad)(dst_r)

  R = 1024
  grid = (npad // R,)
  row_spec = pl.BlockSpec((R, din), lambda r: (r, 0))
  full_spec = pl.BlockSpec((din, dh), lambda r: (0, 0))

  x0 = pl.pallas_call(
      _mm_body,
      grid=grid,
      in_specs=[
          row_spec,
          full_spec,
          pl.BlockSpec((1, dh), lambda r: (0, 0)),
      ],
      out_specs=pl.BlockSpec((R, dh), lambda r: (r, 0)),
      out_shape=jax.ShapeDtypeStruct((npad, dh), F32),
  )(xp, Wh, bh2)

  deg_t = deg.T
  hp, dinv = pl.pallas_call(
      _scale_body,
      grid=grid,
      in_specs=[
          pl.BlockSpec((R, dh), lambda r: (r, 0)),
          pl.BlockSpec((R, 2), lambda r: (r, 0)),
      ],
      out_specs=[pl.BlockSpec((R, dh), lambda r: (r, 0))] * 2,
      out_shape=[jax.ShapeDtypeStruct((npad, dh), F32)] * 2,
  )(x0, deg_t)

  agg_call = _make_agg_kernel(chunks, npad, dh)
  layer_call = pl.pallas_call(
      _layer_body,
      grid=grid,
      in_specs=[
          pl.BlockSpec((NC, R, dh), lambda r: (0, r, 0)),
          pl.BlockSpec((R, dh), lambda r: (r, 0)),
          pl.BlockSpec((R, dh), lambda r: (r, 0)),
          pl.BlockSpec((R, dh), lambda r: (r, 0)),
          pl.BlockSpec((dh, dh), lambda r: (0, 0)),
      ],
      out_specs=[pl.BlockSpec((R, dh), lambda r: (r, 0))] * 2,
      out_shape=[jax.ShapeDtypeStruct((npad, dh), F32)] * 2,
  )

  h = x0
  for w1 in (W1_0, W1_1, W1_2):
    raw = agg_call(hp, src_r, dst_r)
    h, hp = layer_call(raw, h, x0, dinv, w1)

  raw = agg_call(hp, src_r, dst_r)
  y = pl.pallas_call(
      _final_body,
      grid=grid,
      in_specs=[
          pl.BlockSpec((NC, R, dh), lambda r: (0, r, 0)),
          pl.BlockSpec((R, dh), lambda r: (r, 0)),
          pl.BlockSpec((R, dh), lambda r: (r, 0)),
          pl.BlockSpec((R, dh), lambda r: (r, 0)),
          pl.BlockSpec((dh, dh), lambda r: (0, 0)),
          pl.BlockSpec((dh, dout), lambda r: (0, 0)),
          pl.BlockSpec((1, dout), lambda r: (0, 0)),
      ],
      out_specs=pl.BlockSpec((R, dout), lambda r: (r, 0)),
      out_shape=jax.ShapeDtypeStruct((npad, dout), F32),
  )(raw, h, x0, dinv, W1_3, Wo, bo2)

  return y[:n]
